# Initial kernel scaffold; baseline (speedup 1.0000x reference)
import jax
import jax.numpy as jnp
from jax.experimental import pallas as pl


def kernel(x, edge_index, W1, b1, W2, b2):
    raise NotImplementedError("write your pallas kernel here")


# speedup vs baseline: 20.0225x; 20.0225x over previous
import functools

import jax
import jax.numpy as jnp
from jax import lax
from jax.experimental import pallas as pl
from jax.experimental.pallas import tpu as pltpu
from jax.experimental.pallas import tpu_sc as plsc

N = 10000
E = 320000
D = 128

NC = 2
NS = 16
NW = NC * NS
EPW = E // NW

N_PAD = 10240
DNPT = N_PAD // NS
DCHUNK = 2000
DNCH = EPW // DCHUNK

CHUNK = 200
NCH = EPW // CHUNK
NPT = N_PAD // NS

_mesh = plsc.VectorSubcoreMesh(core_axis_name="c", subcore_axis_name="s")


@functools.partial(
    pl.kernel,
    mesh=_mesh,
    out_type=jax.ShapeDtypeStruct((NC, N_PAD), jnp.float32),
    scratch_types=[
        pltpu.VMEM((DCHUNK,), jnp.int32),
        pltpu.VMEM((DCHUNK,), jnp.float32),
        pltpu.VMEM_SHARED((N_PAD,), jnp.float32),
    ],
)
def _sc_degree(col_hbm, ones_hbm, zeros_hbm, out_hbm, cidx, ones_v, acc):
    c = lax.axis_index("c")
    s = lax.axis_index("s")
    wid = s * NC + c
    pltpu.sync_copy(ones_hbm, ones_v)
    pltpu.sync_copy(zeros_hbm.at[pl.ds(s * DNPT, DNPT)],
                    acc.at[pl.ds(s * DNPT, DNPT)])
    plsc.subcore_barrier()

    def body(i, carry):
        off = wid * EPW + i * DCHUNK
        pltpu.sync_copy(col_hbm.at[pl.ds(off, DCHUNK)], cidx)
        pltpu.sync_copy(ones_v, acc.at[cidx], add=True)
        return carry

    lax.fori_loop(0, DNCH, body, 0)
    plsc.subcore_barrier()
    pltpu.sync_copy(acc.at[pl.ds(s * DNPT, DNPT)],
                    out_hbm.at[c, pl.ds(s * DNPT, DNPT)])


@functools.partial(
    pl.kernel,
    mesh=_mesh,
    out_type=jax.ShapeDtypeStruct((NC, N_PAD, D), jnp.float32),
    scratch_types=[
        pltpu.VMEM((CHUNK,), jnp.int32),
        pltpu.VMEM((CHUNK,), jnp.int32),
        pltpu.VMEM((CHUNK, D), jnp.float32),
        pltpu.VMEM_SHARED((N_PAD, D), jnp.float32),
        pltpu.SemaphoreType.DMA,
    ],
)
def _sc_edge_pass(g_hbm, row_hbm, col_hbm, zeros_hbm, out_hbm,
                  ridx, cidx, buf, acc, sem):
    c = lax.axis_index("c")
    s = lax.axis_index("s")
    wid = s * NC + c
    pltpu.sync_copy(zeros_hbm.at[pl.ds(s * NPT, NPT)],
                    acc.at[pl.ds(s * NPT, NPT)])
    plsc.subcore_barrier()

    def body(i, carry):
        off = wid * EPW + i * CHUNK
        pltpu.sync_copy(row_hbm.at[pl.ds(off, CHUNK)], ridx)
        pltpu.sync_copy(col_hbm.at[pl.ds(off, CHUNK)], cidx)
        pltpu.async_copy(g_hbm.at[ridx], buf, sem).wait()
        pltpu.sync_copy(buf, acc.at[cidx], add=True)
        return carry

    lax.fori_loop(0, NCH, body, 0)
    plsc.subcore_barrier()
    pltpu.sync_copy(acc.at[pl.ds(s * NPT, NPT)],
                    out_hbm.at[c, pl.ds(s * NPT, NPT)])


def _tc_mm1(x, W1):
    def body(x_ref, w_ref, h_ref):
        h_ref[...] = jnp.dot(x_ref[...], w_ref[...],
                             preferred_element_type=jnp.float32)

    return pl.pallas_call(
        body, out_shape=jax.ShapeDtypeStruct((N, D), jnp.float32))(x, W1)


def _tc_scale1(h1, dega, degb):
    def body(h_ref, da_ref, db_ref, dis_ref, g_ref):
        deg = da_ref[...] + db_ref[...] + 1.0
        dis = lax.rsqrt(deg)
        dis_ref[...] = dis
        g_ref[...] = h_ref[...] * dis

    return pl.pallas_call(
        body,
        out_shape=(jax.ShapeDtypeStruct((N, 1), jnp.float32),
                   jax.ShapeDtypeStruct((N, D), jnp.float32)))(h1, dega, degb)


def _tc_layer2(s1a, s1b, g1, dis, b1, W2):
    def body(sa_ref, sb_ref, g_ref, dis_ref, b_ref, w_ref, g2_ref):
        z = dis_ref[...] * (sa_ref[...] + sb_ref[...] + g_ref[...]) + b_ref[...]
        z = jnp.maximum(z, 0.0)
        h2 = jnp.dot(z, w_ref[...], preferred_element_type=jnp.float32)
        g2_ref[...] = h2 * dis_ref[...]

    return pl.pallas_call(
        body,
        out_shape=jax.ShapeDtypeStruct((N, D), jnp.float32))(
            s1a, s1b, g1, dis, b1, W2)


def _tc_out(s2a, s2b, g2, dis, b2):
    def body(sa_ref, sb_ref, g_ref, dis_ref, b_ref, o_ref):
        o_ref[...] = (dis_ref[...] * (sa_ref[...] + sb_ref[...] + g_ref[...])
                      + b_ref[...])

    return pl.pallas_call(
        body,
        out_shape=jax.ShapeDtypeStruct((N, D), jnp.float32))(
            s2a, s2b, g2, dis, b2)


def kernel(x, edge_index, W1, b1, W2, b2):
    row = edge_index[0]
    col = edge_index[1]
    ones_e = jnp.ones((DCHUNK,), jnp.float32)
    zeros_1d = jnp.zeros((N_PAD,), jnp.float32)
    zeros_2d = jnp.zeros((N_PAD, D), jnp.float32)

    deg_p = _sc_degree(col, ones_e, zeros_1d)
    h1 = _tc_mm1(x, W1)
    dega = deg_p[0, :N].reshape(N, 1)
    degb = deg_p[1, :N].reshape(N, 1)
    dis, g1 = _tc_scale1(h1, dega, degb)

    s1 = _sc_edge_pass(g1, row, col, zeros_2d)
    g2 = _tc_layer2(s1[0, :N], s1[1, :N], g1, dis, b1.reshape(1, D), W2)

    s2 = _sc_edge_pass(g2, row, col, zeros_2d)
    out = _tc_out(s2[0, :N], s2[1, :N], g2, dis, b2.reshape(1, D))
    return out
